# baseline (speedup 1.0000x reference)
import functools

import jax
import jax.numpy as jnp
from jax import lax
from jax.experimental import pallas as pl
from jax.experimental.pallas import tpu as pltpu
from jax.experimental.pallas import tpu_sc as plsc

B = 16384
F = 64
NC = 2
NS = 16
NW = NC * NS
BPW = B // NW
CHUNK = 128
NCHUNK = BPW // CHUNK


def _mf_body(u_idx_hbm, i_idx_hbm, u_emb_hbm, i_emb_hbm, u_bias_hbm,
             i_bias_hbm, out_hbm,
             uidx_v, iidx_v, u_rows, i_rows, ubval, ibval, out_v, sem):
    cid = lax.axis_index("c")
    sid = lax.axis_index("s")
    wid = sid * NC + cid
    base = wid * BPW

    pltpu.sync_copy(u_idx_hbm.at[pl.ds(base, BPW)], uidx_v)
    pltpu.sync_copy(i_idx_hbm.at[pl.ds(base, BPW)], iidx_v)

    copies = []
    for c in range(NCHUNK):
        sl = pl.ds(c * CHUNK, CHUNK)
        copies.append(pltpu.async_copy(
            u_emb_hbm.at[uidx_v.at[sl]], u_rows.at[sl], sem))
        copies.append(pltpu.async_copy(
            i_emb_hbm.at[iidx_v.at[sl]], i_rows.at[sl], sem))
        copies.append(pltpu.async_copy(
            u_bias_hbm.at[uidx_v.at[sl]], ubval.at[sl], sem))
        copies.append(pltpu.async_copy(
            i_bias_hbm.at[iidx_v.at[sl]], ibval.at[sl], sem))
    for cp in copies:
        cp.wait()

    lane = lax.iota(jnp.int32, 16)

    def body(g, carry):
        sl = pl.ds(g * 16, 16)
        res = ubval[sl] + ibval[sl]
        for j in range(16):
            b = g * 16 + j
            acc = u_rows[b, pl.ds(0, 16)] * i_rows[b, pl.ds(0, 16)]
            for c in range(1, F // 16):
                acc = acc + u_rows[b, pl.ds(c * 16, 16)] * i_rows[b, pl.ds(c * 16, 16)]
            res = res + jnp.where(lane == j, jnp.sum(acc), 0.0)
        out_v[sl] = res
        return carry

    lax.fori_loop(0, BPW // 16, body, 0)

    pltpu.sync_copy(out_v, out_hbm.at[pl.ds(base, BPW)])


def _mf(u_idx, i_idx, u_emb, i_emb, u_bias1, i_bias1):
    mesh = plsc.VectorSubcoreMesh(core_axis_name="c", subcore_axis_name="s")
    f = functools.partial(
        pl.kernel,
        out_type=jax.ShapeDtypeStruct((B,), jnp.float32),
        mesh=mesh,
        scratch_types=[
            pltpu.VMEM((BPW,), jnp.int32),
            pltpu.VMEM((BPW,), jnp.int32),
            pltpu.VMEM((BPW, F), jnp.float32),
            pltpu.VMEM((BPW, F), jnp.float32),
            pltpu.VMEM((BPW,), jnp.float32),
            pltpu.VMEM((BPW,), jnp.float32),
            pltpu.VMEM((BPW,), jnp.float32),
            pltpu.SemaphoreType.DMA,
        ],
        compiler_params=pltpu.CompilerParams(
            needs_layout_passes=False, use_tc_tiling_on_sc=False),
    )(_mf_body)
    return f(u_idx, i_idx, u_emb, i_emb, u_bias1, i_bias1)


def kernel(u_idx, i_idx, u_emb, i_emb, u_bias, i_bias):
    u_bias1 = jnp.sum(u_bias, axis=1)
    i_bias1 = jnp.sum(i_bias, axis=1)
    return _mf(u_idx.astype(jnp.int32), i_idx.astype(jnp.int32),
               u_emb, i_emb, u_bias1, i_bias1)


# speedup vs baseline: 1.4684x; 1.4684x over previous
import functools

import jax
import jax.numpy as jnp
from jax import lax
from jax.experimental import pallas as pl
from jax.experimental.pallas import tpu as pltpu
from jax.experimental.pallas import tpu_sc as plsc

B = 16384
F = 64
NC = 2
NS = 16
NW = NC * NS
BPW = B // NW
Q = 64
NCH = BPW // Q
QG = Q // 16
BCH = 128
NBCH = BPW // BCH


def _mf_body(u_idx_hbm, i_idx_hbm, u_emb_hbm, i_emb_hbm, u_bias_hbm,
             i_bias_hbm, out_hbm,
             uidx_v, iidx_v, u_rowsA, i_rowsA, u_rowsB, i_rowsB,
             ubval, ibval, out_v, semA, semB, semb):
    cid = lax.axis_index("c")
    sid = lax.axis_index("s")
    wid = sid * NC + cid
    base = wid * BPW

    pltpu.sync_copy(u_idx_hbm.at[pl.ds(base, BPW)], uidx_v)
    pltpu.sync_copy(i_idx_hbm.at[pl.ds(base, BPW)], iidx_v)

    bias_copies = []
    for c in range(NBCH):
        sl = pl.ds(c * BCH, BCH)
        bias_copies.append(pltpu.async_copy(
            u_bias_hbm.at[uidx_v.at[sl]], ubval.at[sl], semb))
        bias_copies.append(pltpu.async_copy(
            i_bias_hbm.at[iidx_v.at[sl]], ibval.at[sl], semb))

    lane = lax.iota(jnp.int32, 16)
    ring = ((u_rowsA, i_rowsA, semA), (u_rowsB, i_rowsB, semB))

    def fire(q, bufs):
        u_rows, i_rows, sem = bufs

        def fire_group(g, carry):
            iv_u = uidx_v[pl.ds(q * Q + g * 16, 16)]
            iv_i = iidx_v[pl.ds(q * Q + g * 16, 16)]
            for j in range(16):
                bm = g * 16 + j
                pltpu.async_copy(u_emb_hbm.at[pl.ds(iv_u[j], 1), :],
                                 u_rows.at[pl.ds(bm, 1), :], sem)
                pltpu.async_copy(i_emb_hbm.at[pl.ds(iv_i[j], 1), :],
                                 i_rows.at[pl.ds(bm, 1), :], sem)
            return carry

        lax.fori_loop(0, QG, fire_group, 0)

    def drain(bufs):
        u_rows, i_rows, sem = bufs
        pltpu.make_async_copy(u_emb_hbm.at[pl.ds(0, Q), :],
                              u_rows, sem).wait()
        pltpu.make_async_copy(u_emb_hbm.at[pl.ds(0, Q), :],
                              i_rows, sem).wait()

    def compute(q, bufs):
        u_rows, i_rows, sem = bufs

        def compute_group(g, carry):
            sl = pl.ds(q * Q + g * 16, 16)
            res = ubval[sl] + ibval[sl]
            for j in range(16):
                bm = g * 16 + j
                acc = (u_rows[bm, pl.ds(0, 16)]
                       * i_rows[bm, pl.ds(0, 16)])
                for c in range(1, F // 16):
                    acc = acc + (u_rows[bm, pl.ds(c * 16, 16)]
                                 * i_rows[bm, pl.ds(c * 16, 16)])
                res = res + jnp.where(lane == j, jnp.sum(acc), 0.0)
            out_v[sl] = res
            return carry

        lax.fori_loop(0, QG, compute_group, 0)

    fire(0, ring[0])
    fire(1, ring[1])
    for cp in bias_copies:
        cp.wait()
    for q in range(NCH):
        bufs = ring[q % 2]
        drain(bufs)
        compute(q, bufs)
        if q + 2 < NCH:
            fire(q + 2, bufs)

    pltpu.sync_copy(out_v, out_hbm.at[pl.ds(base, BPW)])


def _mf(u_idx, i_idx, u_emb, i_emb, u_bias1, i_bias1):
    mesh = plsc.VectorSubcoreMesh(core_axis_name="c", subcore_axis_name="s")
    f = functools.partial(
        pl.kernel,
        out_type=jax.ShapeDtypeStruct((B,), jnp.float32),
        mesh=mesh,
        scratch_types=[
            pltpu.VMEM((BPW,), jnp.int32),
            pltpu.VMEM((BPW,), jnp.int32),
            pltpu.VMEM((Q, F), jnp.float32),
            pltpu.VMEM((Q, F), jnp.float32),
            pltpu.VMEM((Q, F), jnp.float32),
            pltpu.VMEM((Q, F), jnp.float32),
            pltpu.VMEM((BPW,), jnp.float32),
            pltpu.VMEM((BPW,), jnp.float32),
            pltpu.VMEM((BPW,), jnp.float32),
            pltpu.SemaphoreType.DMA,
            pltpu.SemaphoreType.DMA,
            pltpu.SemaphoreType.DMA,
        ],
        compiler_params=pltpu.CompilerParams(
            needs_layout_passes=False, use_tc_tiling_on_sc=True),
    )(_mf_body)
    return f(u_idx, i_idx, u_emb, i_emb, u_bias1, i_bias1)


def kernel(u_idx, i_idx, u_emb, i_emb, u_bias, i_bias):
    u_bias1 = jnp.sum(u_bias, axis=1)
    i_bias1 = jnp.sum(i_bias, axis=1)
    return _mf(u_idx.astype(jnp.int32), i_idx.astype(jnp.int32),
               u_emb, i_emb, u_bias1, i_bias1)
